# baseline (speedup 1.0000x reference)
import functools

import jax
import jax.numpy as jnp
from jax import lax
from jax.experimental import pallas as pl
from jax.experimental.pallas import tpu as pltpu
from jax.experimental.pallas import tpu_sc as plsc

EMB = 64
NEG = 3
LANES = 16
NC = 2
NS = 16
NW = NC * NS
GTC = 4
BW = GTC * 128
WIN = 520
G2 = 64

_CP = pltpu.CompilerParams(needs_layout_passes=False, use_tc_tiling_on_sc=True)


def _cdiv(a, b):
    return (a + b - 1) // b


def _mesh():
    return plsc.VectorSubcoreMesh(
        core_axis_name="c", subcore_axis_name="s", num_cores=NC, num_subcores=NS
    )


def _grp_bounds(sidx, ntc, tcpw, ng, vocab):
    w = jnp.arange(NW, dtype=jnp.int32)[:, None]
    g = jnp.arange(ng, dtype=jnp.int32)[None, :]
    hi_tc = jnp.minimum(w * tcpw + (g + 1) * GTC,
                        jnp.minimum((w + 1) * tcpw, ntc))
    his = jnp.minimum(hi_tc * 128, vocab).reshape(-1)
    ends = jnp.searchsorted(sidx, his, side="left").astype(jnp.int32)
    bnds = jnp.concatenate([jnp.zeros((1,), jnp.int32), ends])
    pad = jnp.zeros((128,), jnp.int32) + jnp.int32(sidx.shape[0])
    return jnp.concatenate([bnds, pad])


def _sc_extract(ieT, oeT, st_idx, st_pos, so_idx, so_pos, bnds_t, bnds_o):
    B = st_idx.shape[0]
    NO = so_idx.shape[0]
    V_I, V_O = ieT.shape[1], oeT.shape[1]
    NTC_I, NTC_O = _cdiv(V_I, 128), _cdiv(V_O, 128)
    TCPW_I, TCPW_O = _cdiv(NTC_I, NW), _cdiv(NTC_O, NW)
    NG_I, NG_O = _cdiv(TCPW_I, GTC), _cdiv(TCPW_O, GTC)

    @functools.partial(
        pl.kernel,
        out_type=(
            jax.ShapeDtypeStruct((B + 16, EMB), jnp.float32),
            jax.ShapeDtypeStruct((NO + 16, EMB), jnp.float32),
        ),
        mesh=_mesh(),
        compiler_params=_CP,
        scratch_types=[
            pltpu.VMEM((96,), jnp.int32),
            pltpu.VMEM((WIN,), jnp.int32),
            pltpu.VMEM((WIN,), jnp.int32),
            pltpu.VMEM((EMB, BW), jnp.float32),
            pltpu.VMEM((2 * LANES, EMB), jnp.float32),
            pltpu.SemaphoreType.DMA,
            pltpu.SemaphoreType.DMA,
            pltpu.SemaphoreType.DMA,
        ],
    )
    def extract(ie_hbm, oe_hbm, sti_hbm, stp_hbm, soi_hbm, sop_hbm,
                bt_hbm, bo_hbm, tr_out, cn_out,
                bvm, win_i, win_p, buf, stg, sem_d, sem_w, sem_io):
        wid = lax.axis_index("s") * NC + lax.axis_index("c")
        rows_e = [lax.iota(jnp.int32, LANES) + q * LANES for q in range(4)]

        def drain16(par16):
            pltpu.make_async_copy(
                tr_out.at[pl.ds(0, LANES), :],
                stg.at[pl.ds(par16, LANES), :], sem_w).wait()

        def phase(tab, sidx, spos, bnds, rows_out, ntc, tcpw, ng, nitems,
                  dummy_row, tc0):
            a0 = pl.multiple_of((wid * ng) & -8, 8)
            off = wid * ng - a0
            pltpu.sync_copy(bnds.at[pl.ds(a0, 88)], bvm.at[pl.ds(0, 88)])

            def grp(g, tc_c):
                bv = bvm[pl.ds(off + g, LANES)]
                seg_lo, seg_hi = bv[0], bv[1]
                cnt = seg_hi - seg_lo
                lo_tc = wid * tcpw + g * GTC
                dma_tc = jnp.minimum(lo_tc, ntc - GTC)
                for tr in range(8):
                    pltpu.async_copy(
                        tab.at[pl.ds(tr * 8, 8), pl.ds(dma_tc * 128, BW)],
                        buf.at[pl.ds(tr * 8, 8), :], sem_d)
                pltpu.make_async_copy(
                    tab.at[pl.ds(0, EMB), pl.ds(0, BW)], buf, sem_d).wait()
                col0 = dma_tc * 128
                n_win = (cnt + 511) >> 9

                def win_loop(wi, tc_w):
                    wstart = seg_lo + wi * 512
                    wcnt = jnp.minimum(cnt - wi * 512, 512)
                    ws = pl.multiple_of(
                        jnp.minimum(wstart, nitems - WIN) & -8, 8)
                    pltpu.sync_copy(sidx.at[pl.ds(ws, WIN)], win_i)
                    pltpu.sync_copy(spos.at[pl.ds(ws, WIN)], win_p)
                    loff = wstart - ws
                    n_chunks = (wcnt + 15) >> 4

                    def chunk(c, tc_cc):
                        base = loff + c * LANES
                        v = win_i[pl.ds(base, LANES)]
                        pp = win_p[pl.ds(base, LANES)]
                        rem = wcnt - c * LANES
                        par16 = (tc_cc & 1) * LANES

                        @pl.when(tc_cc >= 2)
                        def _():
                            drain16(par16)

                        for ii in range(LANES):
                            valid = ii < rem
                            col = jnp.where(valid, v[ii] - col0, 0)
                            posw = jnp.where(valid, pp[ii], dummy_row)
                            cs = jnp.zeros((LANES,), jnp.int32) + col
                            srow = par16 + ii
                            for q in range(4):
                                stg[srow, pl.ds(q * LANES, LANES)] = (
                                    plsc.load_gather(buf, [rows_e[q], cs]))
                            pltpu.async_copy(
                                stg.at[srow, :], rows_out.at[posw, :], sem_w)
                        return tc_cc + 1

                    return lax.fori_loop(0, n_chunks, chunk, tc_w)

                return lax.fori_loop(0, n_win, win_loop, tc_c)

            return lax.fori_loop(0, ng, grp, tc0)

        tc1 = phase(ie_hbm, sti_hbm, stp_hbm, bt_hbm, tr_out,
                    NTC_I, TCPW_I, NG_I, B, B, 0)
        tc2 = phase(oe_hbm, soi_hbm, sop_hbm, bo_hbm, cn_out,
                    NTC_O, TCPW_O, NG_O, NO, NO, tc1)

        @pl.when(tc2 >= 1)
        def _():
            drain16(((tc2 - 1) & 1) * LANES)

        @pl.when(tc2 >= 2)
        def _():
            drain16(((tc2 - 2) & 1) * LANES)

    return extract(ieT, oeT, st_idx, st_pos, so_idx, so_pos, bnds_t, bnds_o)


def _sc_scores(t_rows, cn_rows, B):
    per_w = B // NW
    n_groups = per_w // G2

    @functools.partial(
        pl.kernel,
        out_type=(
            jax.ShapeDtypeStruct((B,), jnp.float32),
            jax.ShapeDtypeStruct((B,), jnp.float32),
        ),
        mesh=_mesh(),
        compiler_params=_CP,
        scratch_types=[
            pltpu.VMEM((G2, EMB), jnp.float32),
            pltpu.VMEM((G2, EMB), jnp.float32),
            pltpu.VMEM((NEG * G2, EMB), jnp.float32),
            pltpu.VMEM((G2 * LANES,), jnp.float32),
            pltpu.VMEM((G2 * LANES,), jnp.float32),
            pltpu.VMEM((G2,), jnp.float32),
            pltpu.VMEM((G2,), jnp.float32),
            pltpu.SemaphoreType.DMA,
        ],
    )
    def scores(tr_hbm, cn_hbm, pos_hbm, neg_hbm,
               tb, cb, nb, pv_buf, nv_buf, pos_buf, neg_buf, sem):
        wid = lax.axis_index("s") * NC + lax.axis_index("c")
        wbase = wid * per_w
        iota16 = lax.iota(jnp.int32, LANES)

        def group(g, _):
            base = pl.multiple_of(wbase + g * G2, 8)

            def fire(sg, _):
                ib = sg * LANES
                for ii in range(LANES):
                    i = ib + ii
                    pltpu.async_copy(
                        tr_hbm.at[base + i, :], tb.at[i, :], sem)
                    pltpu.async_copy(
                        cn_hbm.at[base + i, :], cb.at[i, :], sem)
                    for k in range(NEG):
                        pltpu.async_copy(
                            cn_hbm.at[B + NEG * (base + i) + k, :],
                            nb.at[NEG * i + k, :], sem)
                return 0

            lax.fori_loop(0, G2 // LANES, fire, 0)
            pltpu.make_async_copy(
                tr_hbm.at[pl.ds(0, G2), :], tb, sem).wait()
            pltpu.make_async_copy(
                tr_hbm.at[pl.ds(0, G2), :], cb, sem).wait()
            pltpu.make_async_copy(
                tr_hbm.at[pl.ds(0, NEG * G2), :], nb, sem).wait()

            def item(i, _):
                t0 = tb[i, pl.ds(0, LANES)]
                t1 = tb[i, pl.ds(LANES, LANES)]
                t2 = tb[i, pl.ds(2 * LANES, LANES)]
                t3 = tb[i, pl.ds(3 * LANES, LANES)]
                pv = (t0 * cb[i, pl.ds(0, LANES)]
                      + t1 * cb[i, pl.ds(LANES, LANES)]
                      + t2 * cb[i, pl.ds(2 * LANES, LANES)]
                      + t3 * cb[i, pl.ds(3 * LANES, LANES)])
                pv_buf[pl.ds(i * LANES, LANES)] = pv
                j = i * NEG
                nv = (t0 * nb[j, pl.ds(0, LANES)]
                      + t1 * nb[j, pl.ds(LANES, LANES)]
                      + t2 * nb[j, pl.ds(2 * LANES, LANES)]
                      + t3 * nb[j, pl.ds(3 * LANES, LANES)])
                nv += (t0 * nb[j + 1, pl.ds(0, LANES)]
                       + t1 * nb[j + 1, pl.ds(LANES, LANES)]
                       + t2 * nb[j + 1, pl.ds(2 * LANES, LANES)]
                       + t3 * nb[j + 1, pl.ds(3 * LANES, LANES)])
                nv += (t0 * nb[j + 2, pl.ds(0, LANES)]
                       + t1 * nb[j + 2, pl.ds(LANES, LANES)]
                       + t2 * nb[j + 2, pl.ds(2 * LANES, LANES)]
                       + t3 * nb[j + 2, pl.ds(3 * LANES, LANES)])
                nv_buf[pl.ds(i * LANES, LANES)] = nv
                return 0

            lax.fori_loop(0, G2, item, 0)

            def red(jg, _):
                rows = (jg * LANES + iota16) * LANES
                accp = plsc.load_gather(pv_buf, [rows])
                accn = plsc.load_gather(nv_buf, [rows])
                for l in range(1, LANES):
                    accp += plsc.load_gather(pv_buf, [rows + l])
                    accn += plsc.load_gather(nv_buf, [rows + l])
                pos_buf[pl.ds(jg * LANES, LANES)] = accp
                neg_buf[pl.ds(jg * LANES, LANES)] = -accn
                return 0

            lax.fori_loop(0, G2 // LANES, red, 0)
            pltpu.sync_copy(pos_buf, pos_hbm.at[pl.ds(base, G2)])
            pltpu.sync_copy(neg_buf, neg_hbm.at[pl.ds(base, G2)])
            return 0

        lax.fori_loop(0, n_groups, group, 0)

    return scores(t_rows, cn_rows)


def _tc_loss(pos, neg):
    B = pos.shape[0]
    p2 = pos.reshape(B // 128, 128)
    n2 = neg.reshape(B // 128, 128)

    def body(p_ref, n_ref, o_ref):
        x = jax.nn.log_sigmoid(p_ref[...]) + jax.nn.log_sigmoid(n_ref[...])
        o_ref[0, 0] = -jnp.sum(x) / B

    out = pl.pallas_call(
        body,
        out_shape=jax.ShapeDtypeStruct((1, 1), jnp.float32),
        out_specs=pl.BlockSpec(memory_space=pltpu.SMEM),
    )(p2, n2)
    return out[0, 0]


def kernel(targets, contexts, negsamples, device, in_emb, out_emb):
    del device
    B = targets.shape[0]
    tg = targets.astype(jnp.int32)
    cx = contexts.astype(jnp.int32)
    ng = negsamples.astype(jnp.int32)

    st_idx, st_pos = lax.sort_key_val(tg, jnp.arange(B, dtype=jnp.int32))
    oidx = jnp.concatenate([cx, ng])
    NO = oidx.shape[0]
    so_idx, so_pos = lax.sort_key_val(oidx, jnp.arange(NO, dtype=jnp.int32))

    V_I, V_O = in_emb.shape[0], out_emb.shape[0]
    NTC_I, NTC_O = _cdiv(V_I, 128), _cdiv(V_O, 128)
    bnds_t = _grp_bounds(st_idx, NTC_I, _cdiv(NTC_I, NW),
                         _cdiv(_cdiv(NTC_I, NW), GTC), V_I)
    bnds_o = _grp_bounds(so_idx, NTC_O, _cdiv(NTC_O, NW),
                         _cdiv(_cdiv(NTC_O, NW), GTC), V_O)

    t_rows, cn_rows = _sc_extract(
        in_emb.T, out_emb.T, st_idx, st_pos, so_idx, so_pos, bnds_t, bnds_o)
    pos, neg = _sc_scores(t_rows, cn_rows, B)
    return _tc_loss(pos, neg)


# speedup vs baseline: 3.0652x; 3.0652x over previous
import functools

import jax
import jax.numpy as jnp
from jax import lax
from jax.experimental import pallas as pl
from jax.experimental.pallas import tpu as pltpu
from jax.experimental.pallas import tpu_sc as plsc

EMB = 64
NEG = 3
LANES = 16
NC = 2
NS = 16
NW = NC * NS
G = 64


def _sc_scores(targets, contexts, negsamples, in_emb, out_emb):
    B = targets.shape[0]
    per_w = B // NW
    n_groups = per_w // G
    mesh = plsc.VectorSubcoreMesh(
        core_axis_name="c", subcore_axis_name="s", num_cores=NC, num_subcores=NS
    )

    @functools.partial(
        pl.kernel,
        out_type=(
            jax.ShapeDtypeStruct((B,), jnp.float32),
            jax.ShapeDtypeStruct((B,), jnp.float32),
        ),
        mesh=mesh,
        compiler_params=pltpu.CompilerParams(
            needs_layout_passes=False, use_tc_tiling_on_sc=True
        ),
        scratch_types=[
            pltpu.VMEM((per_w,), jnp.int32),
            pltpu.VMEM((per_w,), jnp.int32),
            pltpu.VMEM((NEG * per_w,), jnp.int32),
            pltpu.VMEM((G, EMB), jnp.float32),
            pltpu.VMEM((G, EMB), jnp.float32),
            pltpu.VMEM((G, EMB), jnp.float32),
            pltpu.VMEM((G, EMB), jnp.float32),
            pltpu.VMEM((NEG * G, EMB), jnp.float32),
            pltpu.VMEM((NEG * G, EMB), jnp.float32),
            pltpu.VMEM((G * LANES,), jnp.float32),
            pltpu.VMEM((G * LANES,), jnp.float32),
            pltpu.VMEM((G,), jnp.float32),
            pltpu.VMEM((G,), jnp.float32),
            pltpu.SemaphoreType.DMA,
            pltpu.SemaphoreType.DMA,
            pltpu.SemaphoreType.DMA,
        ],
    )
    def scores(tg_hbm, cx_hbm, ng_hbm, ie_hbm, oe_hbm, pos_hbm, neg_hbm,
               idx_t, idx_c, idx_n, t0_buf, t1_buf, c0_buf, c1_buf,
               n0_buf, n1_buf, pv_buf, nv_buf, pos_buf, neg_buf,
               sem0, sem1, sem_io):
        wid = lax.axis_index("s") * NC + lax.axis_index("c")
        wbase = wid * per_w

        t_bufs = (t0_buf, t1_buf)
        c_bufs = (c0_buf, c1_buf)
        n_bufs = (n0_buf, n1_buf)
        sems = (sem0, sem1)

        cp1 = pltpu.async_copy(tg_hbm.at[pl.ds(wbase, per_w)], idx_t, sem_io)
        cp2 = pltpu.async_copy(cx_hbm.at[pl.ds(wbase, per_w)], idx_c, sem_io)
        cp3 = pltpu.async_copy(
            ng_hbm.at[pl.ds(NEG * wbase, NEG * per_w)], idx_n, sem_io)
        cp1.wait()
        cp2.wait()
        cp3.wait()

        def fire(g, b):
            tb, cb, nb, sem = t_bufs[b], c_bufs[b], n_bufs[b], sems[b]

            def sub(gi, _):
                ibase = gi * LANES
                tv = idx_t[pl.ds(g * G + ibase, LANES)]
                cv = idx_c[pl.ds(g * G + ibase, LANES)]
                nvs = [idx_n[pl.ds(NEG * (g * G + ibase) + k * LANES, LANES)]
                       for k in range(NEG)]
                for ii in range(LANES):
                    i = ibase + ii
                    pltpu.async_copy(ie_hbm.at[tv[ii], :], tb.at[i, :], sem)
                    pltpu.async_copy(oe_hbm.at[cv[ii], :], cb.at[i, :], sem)
                    for k in range(NEG):
                        j = NEG * ii + k
                        pltpu.async_copy(
                            oe_hbm.at[nvs[j // LANES][j % LANES], :],
                            nb.at[NEG * i + k, :], sem)
                return 0

            lax.fori_loop(0, G // LANES, sub, 0)

        def drain(b):
            tb, cb, nb, sem = t_bufs[b], c_bufs[b], n_bufs[b], sems[b]
            pltpu.make_async_copy(ie_hbm.at[pl.ds(0, G), :], tb, sem).wait()
            pltpu.make_async_copy(oe_hbm.at[pl.ds(0, G), :], cb, sem).wait()
            pltpu.make_async_copy(
                oe_hbm.at[pl.ds(0, NEG * G), :], nb, sem).wait()

        def compute(g, b):
            tb, cb, nb = t_bufs[b], c_bufs[b], n_bufs[b]

            def item(i, _):
                t0 = tb[i, pl.ds(0, LANES)]
                t1 = tb[i, pl.ds(LANES, LANES)]
                t2 = tb[i, pl.ds(2 * LANES, LANES)]
                t3 = tb[i, pl.ds(3 * LANES, LANES)]
                pv = (t0 * cb[i, pl.ds(0, LANES)]
                      + t1 * cb[i, pl.ds(LANES, LANES)]
                      + t2 * cb[i, pl.ds(2 * LANES, LANES)]
                      + t3 * cb[i, pl.ds(3 * LANES, LANES)])
                pv_buf[pl.ds(i * LANES, LANES)] = pv
                j = i * NEG
                nv = (t0 * nb[j, pl.ds(0, LANES)]
                      + t1 * nb[j, pl.ds(LANES, LANES)]
                      + t2 * nb[j, pl.ds(2 * LANES, LANES)]
                      + t3 * nb[j, pl.ds(3 * LANES, LANES)])
                nv += (t0 * nb[j + 1, pl.ds(0, LANES)]
                       + t1 * nb[j + 1, pl.ds(LANES, LANES)]
                       + t2 * nb[j + 1, pl.ds(2 * LANES, LANES)]
                       + t3 * nb[j + 1, pl.ds(3 * LANES, LANES)])
                nv += (t0 * nb[j + 2, pl.ds(0, LANES)]
                       + t1 * nb[j + 2, pl.ds(LANES, LANES)]
                       + t2 * nb[j + 2, pl.ds(2 * LANES, LANES)]
                       + t3 * nb[j + 2, pl.ds(3 * LANES, LANES)])
                nv_buf[pl.ds(i * LANES, LANES)] = nv
                return 0

            lax.fori_loop(0, G, item, 0)

            iota16 = lax.iota(jnp.int32, LANES)

            def red(jg, _):
                rows = (jg * LANES + iota16) * LANES
                accp = plsc.load_gather(pv_buf, [rows])
                accn = plsc.load_gather(nv_buf, [rows])
                for l in range(1, LANES):
                    accp += plsc.load_gather(pv_buf, [rows + l])
                    accn += plsc.load_gather(nv_buf, [rows + l])
                pos_buf[pl.ds(jg * LANES, LANES)] = accp
                neg_buf[pl.ds(jg * LANES, LANES)] = -accn
                return 0

            lax.fori_loop(0, G // LANES, red, 0)
            base = wbase + g * G
            pltpu.sync_copy(pos_buf, pos_hbm.at[pl.ds(base, G)])
            pltpu.sync_copy(neg_buf, neg_hbm.at[pl.ds(base, G)])

        fire(0, 0)

        def pair(gg, _):
            g0 = 2 * gg
            fire(g0 + 1, 1)
            drain(0)
            compute(g0, 0)

            @pl.when(gg < n_groups // 2 - 1)
            def _():
                fire(g0 + 2, 0)

            drain(1)
            compute(g0 + 1, 1)
            return 0

        lax.fori_loop(0, n_groups // 2, pair, 0)

    return scores(targets, contexts, negsamples, in_emb, out_emb)


def _tc_loss(pos, neg):
    B = pos.shape[0]
    p2 = pos.reshape(B // 128, 128)
    n2 = neg.reshape(B // 128, 128)

    def body(p_ref, n_ref, o_ref):
        x = jax.nn.log_sigmoid(p_ref[...]) + jax.nn.log_sigmoid(n_ref[...])
        o_ref[0, 0] = -jnp.sum(x) / B

    out = pl.pallas_call(
        body,
        out_shape=jax.ShapeDtypeStruct((1, 1), jnp.float32),
        out_specs=pl.BlockSpec(memory_space=pltpu.SMEM),
    )(p2, n2)
    return out[0, 0]


def kernel(targets, contexts, negsamples, device, in_emb, out_emb):
    del device
    pos, neg = _sc_scores(
        targets.astype(jnp.int32),
        contexts.astype(jnp.int32),
        negsamples.astype(jnp.int32),
        in_emb,
        out_emb,
    )
    return _tc_loss(pos, neg)
